# baseline (speedup 1.0000x reference)
import functools

import jax
import jax.numpy as jnp
from jax import lax
from jax.experimental import pallas as pl
from jax.experimental.pallas import tpu as pltpu
from jax.experimental.pallas import tpu_sc as plsc

B = 16384
N_DENSE = 13
N_SPARSE = 26
NUM_WORKERS = 32
ROWS_PER_W = B // NUM_WORKERS
LANES = 16
SLICES = ROWS_PER_W // LANES

IDX_PER_W = ROWS_PER_W * N_SPARSE
DEN_PER_W = ROWS_PER_W * N_DENSE


def _body(sparse_hbm, dense_hbm, w_hbm, wdb_hbm, out_hbm,
          idx_v, vals_v, den_v, out_v, wdb_v, sem):
    wid = lax.axis_index("s") * 2 + lax.axis_index("c")

    pltpu.sync_copy(sparse_hbm.at[wid], idx_v)
    pltpu.sync_copy(dense_hbm.at[wid], den_v)
    pltpu.sync_copy(wdb_hbm, wdb_v)

    pltpu.async_copy(w_hbm.at[idx_v], vals_v, sem).wait()

    bdv = wdb_v[pl.ds(N_DENSE * LANES, LANES)]

    def slice_body(i, carry):
        r0 = i * LANES
        s = bdv
        for f in range(N_SPARSE):
            s = s + vals_v[pl.ds(f * ROWS_PER_W + r0, LANES)]
        for j in range(N_DENSE):
            s = s + (wdb_v[pl.ds(j * LANES, LANES)]
                     * den_v[pl.ds(j * ROWS_PER_W + r0, LANES)])
        out_v[pl.ds(r0, LANES)] = s
        return carry

    lax.fori_loop(0, SLICES, slice_body, 0)

    pltpu.sync_copy(out_v, out_hbm.at[pl.ds(wid * ROWS_PER_W, ROWS_PER_W)])


@functools.partial(
    pl.kernel,
    out_type=jax.ShapeDtypeStruct((B,), jnp.float32),
    mesh=plsc.VectorSubcoreMesh(core_axis_name="c", subcore_axis_name="s"),
    scratch_types=[
        pltpu.VMEM((IDX_PER_W,), jnp.int32),
        pltpu.VMEM((IDX_PER_W,), jnp.float32),
        pltpu.VMEM((DEN_PER_W,), jnp.float32),
        pltpu.VMEM((ROWS_PER_W,), jnp.float32),
        pltpu.VMEM(((N_DENSE + 1) * LANES,), jnp.float32),
        pltpu.SemaphoreType.DMA,
    ],
)
def _sc_kernel(sparse_hbm, dense_hbm, w_hbm, wdb_hbm, out_hbm,
               idx_v, vals_v, den_v, out_v, wdb_v, sem):
    _body(sparse_hbm, dense_hbm, w_hbm, wdb_hbm, out_hbm,
          idx_v, vals_v, den_v, out_v, wdb_v, sem)


def kernel(dense_input, sparse_input, w, Wd, bd):
    sparse_t = jnp.transpose(
        sparse_input.astype(jnp.int32).reshape(NUM_WORKERS, ROWS_PER_W, N_SPARSE),
        (0, 2, 1)).reshape(NUM_WORKERS, IDX_PER_W)
    dense_t = jnp.transpose(
        dense_input.reshape(NUM_WORKERS, ROWS_PER_W, N_DENSE),
        (0, 2, 1)).reshape(NUM_WORKERS, DEN_PER_W)
    w_flat = w.reshape(-1)
    wdb = jnp.concatenate(
        [jnp.repeat(Wd.reshape(-1), LANES), jnp.repeat(bd.reshape(-1), LANES)])
    out = _sc_kernel(sparse_t, dense_t, w_flat, wdb)
    return out.reshape(B, 1)


# speedup vs baseline: 1.0691x; 1.0691x over previous
import functools

import jax
import jax.numpy as jnp
from jax import lax
from jax.experimental import pallas as pl
from jax.experimental.pallas import tpu as pltpu
from jax.experimental.pallas import tpu_sc as plsc

B = 16384
N_DENSE = 13
N_SPARSE = 26
VOCAB = 1000000
NUM_CORES = 2
NUM_SUBCORES = 16
NUM_WORKERS = NUM_CORES * NUM_SUBCORES
ROWS_PER_W = B // NUM_WORKERS
LANES = 16
SLICES = ROWS_PER_W // LANES

IDX_PER_W = ROWS_PER_W * N_SPARSE
DEN_PER_W = ROWS_PER_W * N_DENSE

VOCAB_PAD = 1000448
SHARD = VOCAB_PAD // NUM_SUBCORES
N_CHUNK = 8
CHUNK = SHARD // N_CHUNK


def _body(sparse_hbm, dense_hbm, w_hbm, wdb_hbm, out_hbm,
          idx_v, vals_v, den_v, out_v, wdb_v, w_sh, w_tmp, sem, semw):
    cid = lax.axis_index("c")
    sid = lax.axis_index("s")
    wid = sid * NUM_CORES + cid

    pltpu.sync_copy(sparse_hbm.at[wid], idx_v)
    pltpu.sync_copy(dense_hbm.at[wid], den_v)
    pltpu.sync_copy(wdb_hbm, wdb_v)

    sbase = sid * SHARD
    reads = [None, None]
    reads[0] = pltpu.async_copy(
        w_hbm.at[pl.ds(sbase, CHUNK)], w_tmp.at[pl.ds(0, CHUNK)], semw)
    for k in range(N_CHUNK):
        p = k % 2
        if k + 1 < N_CHUNK:
            reads[1 - p] = pltpu.async_copy(
                w_hbm.at[pl.ds(sbase + (k + 1) * CHUNK, CHUNK)],
                w_tmp.at[pl.ds((1 - p) * CHUNK, CHUNK)], semw)
        reads[p].wait()
        pltpu.sync_copy(w_tmp.at[pl.ds(p * CHUNK, CHUNK)],
                        w_sh.at[pl.ds(sbase + k * CHUNK, CHUNK)])
    plsc.subcore_barrier()

    pltpu.async_copy(w_sh.at[idx_v], vals_v, sem).wait()

    bdv = wdb_v[pl.ds(N_DENSE * LANES, LANES)]

    def slice_body(i, carry):
        r0 = i * LANES
        s = bdv
        for f in range(N_SPARSE):
            s = s + vals_v[pl.ds(f * ROWS_PER_W + r0, LANES)]
        for j in range(N_DENSE):
            s = s + (wdb_v[pl.ds(j * LANES, LANES)]
                     * den_v[pl.ds(j * ROWS_PER_W + r0, LANES)])
        out_v[pl.ds(r0, LANES)] = s
        return carry

    lax.fori_loop(0, SLICES, slice_body, 0)

    pltpu.sync_copy(out_v, out_hbm.at[pl.ds(wid * ROWS_PER_W, ROWS_PER_W)])


@functools.partial(
    pl.kernel,
    out_type=jax.ShapeDtypeStruct((B,), jnp.float32),
    mesh=plsc.VectorSubcoreMesh(core_axis_name="c", subcore_axis_name="s"),
    scratch_types=[
        pltpu.VMEM((IDX_PER_W,), jnp.int32),
        pltpu.VMEM((IDX_PER_W,), jnp.float32),
        pltpu.VMEM((DEN_PER_W,), jnp.float32),
        pltpu.VMEM((ROWS_PER_W,), jnp.float32),
        pltpu.VMEM(((N_DENSE + 1) * LANES,), jnp.float32),
        pltpu.VMEM_SHARED((VOCAB_PAD,), jnp.float32),
        pltpu.VMEM((2 * CHUNK,), jnp.float32),
        pltpu.SemaphoreType.DMA,
        pltpu.SemaphoreType.DMA,
    ],
)
def _sc_kernel(sparse_hbm, dense_hbm, w_hbm, wdb_hbm, out_hbm,
               idx_v, vals_v, den_v, out_v, wdb_v, w_sh, w_tmp, sem, semw):
    _body(sparse_hbm, dense_hbm, w_hbm, wdb_hbm, out_hbm,
          idx_v, vals_v, den_v, out_v, wdb_v, w_sh, w_tmp, sem, semw)


def kernel(dense_input, sparse_input, w, Wd, bd):
    sparse_t = jnp.transpose(
        sparse_input.astype(jnp.int32).reshape(NUM_WORKERS, ROWS_PER_W, N_SPARSE),
        (0, 2, 1)).reshape(NUM_WORKERS, IDX_PER_W)
    dense_t = jnp.transpose(
        dense_input.reshape(NUM_WORKERS, ROWS_PER_W, N_DENSE),
        (0, 2, 1)).reshape(NUM_WORKERS, DEN_PER_W)
    w_flat = jnp.pad(w.reshape(-1), (0, VOCAB_PAD - VOCAB))
    wdb = jnp.concatenate(
        [jnp.repeat(Wd.reshape(-1), LANES), jnp.repeat(bd.reshape(-1), LANES)])
    out = _sc_kernel(sparse_t, dense_t, w_flat, wdb)
    return out.reshape(B, 1)


# speedup vs baseline: 1.0729x; 1.0036x over previous
import functools

import jax
import jax.numpy as jnp
from jax import lax
from jax.experimental import pallas as pl
from jax.experimental.pallas import tpu as pltpu
from jax.experimental.pallas import tpu_sc as plsc

B = 16384
N_DENSE = 13
N_SPARSE = 26
VOCAB = 1000000
NUM_CORES = 2
NUM_SUBCORES = 16
NUM_WORKERS = NUM_CORES * NUM_SUBCORES
ROWS_PER_W = B // NUM_WORKERS
LANES = 16
SLICES = ROWS_PER_W // LANES

IDX_PER_W = ROWS_PER_W * N_SPARSE
DEN_PER_W = ROWS_PER_W * N_DENSE

VOCAB_PAD = 1000448
SHARD = VOCAB_PAD // NUM_SUBCORES
N_CHUNK = 8
CHUNK = SHARD // N_CHUNK


def _body(sparse_hbm, dense_hbm, w_hbm, wdb_hbm, out_hbm,
          idx_v, vals_v, den_v, out_v, wdb_v, w_sh, w_tmp, sem, semw):
    cid = lax.axis_index("c")
    sid = lax.axis_index("s")
    wid = sid * NUM_CORES + cid

    pltpu.sync_copy(sparse_hbm.at[wid], idx_v)
    pltpu.sync_copy(dense_hbm.at[wid], den_v)
    pltpu.sync_copy(wdb_hbm, wdb_v)

    sbase = sid * SHARD
    reads = [None, None]
    reads[0] = pltpu.async_copy(
        w_hbm.at[pl.ds(sbase, CHUNK)], w_tmp.at[pl.ds(0, CHUNK)], semw)
    for k in range(N_CHUNK):
        p = k % 2
        if k + 1 < N_CHUNK:
            reads[1 - p] = pltpu.async_copy(
                w_hbm.at[pl.ds(sbase + (k + 1) * CHUNK, CHUNK)],
                w_tmp.at[pl.ds((1 - p) * CHUNK, CHUNK)], semw)
        reads[p].wait()
        pltpu.sync_copy(w_tmp.at[pl.ds(p * CHUNK, CHUNK)],
                        w_sh.at[pl.ds(sbase + k * CHUNK, CHUNK)])
    plsc.subcore_barrier()

    pltpu.async_copy(w_sh.at[idx_v], vals_v, sem).wait()

    bdv = wdb_v[pl.ds(N_DENSE * LANES, LANES)]

    def slice_body(i, carry):
        r0 = i * LANES
        s = bdv
        for f in range(N_SPARSE):
            s = s + vals_v[pl.ds(f * ROWS_PER_W + r0, LANES)]
        for j in range(N_DENSE):
            s = s + (wdb_v[pl.ds(j * LANES, LANES)]
                     * den_v[pl.ds(j * ROWS_PER_W + r0, LANES)])
        out_v[pl.ds(r0, LANES)] = s
        return carry

    lax.fori_loop(0, SLICES, slice_body, 0)

    pltpu.sync_copy(out_v, out_hbm.at[pl.ds(wid * ROWS_PER_W, ROWS_PER_W)])


@functools.partial(
    pl.kernel,
    out_type=jax.ShapeDtypeStruct((B,), jnp.float32),
    mesh=plsc.VectorSubcoreMesh(core_axis_name="c", subcore_axis_name="s"),
    scratch_types=[
        pltpu.VMEM((IDX_PER_W,), jnp.int32),
        pltpu.VMEM((IDX_PER_W,), jnp.float32),
        pltpu.VMEM((DEN_PER_W,), jnp.float32),
        pltpu.VMEM((ROWS_PER_W,), jnp.float32),
        pltpu.VMEM(((N_DENSE + 1) * LANES,), jnp.float32),
        pltpu.VMEM_SHARED((VOCAB_PAD,), jnp.float32),
        pltpu.VMEM((2 * CHUNK,), jnp.float32),
        pltpu.SemaphoreType.DMA,
        pltpu.SemaphoreType.DMA,
    ],
)
def _sc_kernel(sparse_hbm, dense_hbm, w_hbm, wdb_hbm, out_hbm,
               idx_v, vals_v, den_v, out_v, wdb_v, w_sh, w_tmp, sem, semw):
    _body(sparse_hbm, dense_hbm, w_hbm, wdb_hbm, out_hbm,
          idx_v, vals_v, den_v, out_v, wdb_v, w_sh, w_tmp, sem, semw)


def kernel(dense_input, sparse_input, w, Wd, bd):
    sparse_t = jnp.transpose(
        sparse_input.astype(jnp.int32).reshape(NUM_WORKERS, ROWS_PER_W, N_SPARSE),
        (0, 2, 1)).reshape(NUM_WORKERS, IDX_PER_W)
    dense_t = jnp.transpose(
        dense_input.reshape(NUM_WORKERS, ROWS_PER_W, N_DENSE),
        (0, 2, 1)).reshape(NUM_WORKERS, DEN_PER_W)
    w_flat = jnp.pad(w[:, 0], (0, VOCAB_PAD - VOCAB))
    wdb = jnp.concatenate(
        [jnp.repeat(Wd.reshape(-1), LANES), jnp.repeat(bd.reshape(-1), LANES)])
    out = _sc_kernel(sparse_t, dense_t, w_flat, wdb)
    return out.reshape(B, 1)
